# baseline (speedup 1.0000x reference)
import functools

import jax
import jax.numpy as jnp
from jax import lax
from jax.experimental import pallas as pl
from jax.experimental.pallas import tpu as pltpu
from jax.experimental.pallas import tpu_sc as plsc

_D = 32
_NC, _NS = 2, 16
_NW = _NC * _NS
_CHUNK = 128
_NFIRE = 8
_SUPER = _CHUNK * _NFIRE


def _gather_body(ids_hbm, offs_hbm, table_hbm, out_hbm,
                 idx_v, offs_v, buf0, buf1, gsem0, gsem1):
    wid = lax.axis_index("s") * _NC + lax.axis_index("c")
    n_chunks = ids_hbm.shape[1]
    n_super = n_chunks // _NFIRE
    rows_per_w = n_chunks * _CHUNK
    base = wid * rows_per_w

    pltpu.sync_copy(ids_hbm.at[wid], idx_v)
    pltpu.sync_copy(offs_hbm, offs_v)
    off = offs_v[...]

    def _add_off(j, carry):
        for k in range(_CHUNK // 16):
            sl = (j, pl.ds(k * 16, 16))
            idx_v[sl] = idx_v[sl] + off
        return carry
    lax.fori_loop(0, n_chunks, _add_off, 0)

    def _fire(s, buf, sem):
        for k in range(_NFIRE):
            pltpu.async_copy(
                table_hbm.at[idx_v.at[s * _NFIRE + k]],
                buf.at[pl.ds(k * _CHUNK, _CHUNK)],
                sem)

    def _drain(buf, sem):
        for k in range(_NFIRE):
            pltpu.make_async_copy(
                table_hbm.at[idx_v.at[0]],
                buf.at[pl.ds(k * _CHUNK, _CHUNK)],
                sem).wait()

    def _super_pair(s2, carry):
        s = s2 * 2
        _fire(s, buf0, gsem0)
        _drain(buf0, gsem0)
        _fire(s + 1, buf1, gsem1)
        pltpu.sync_copy(buf0, out_hbm.at[pl.ds(base + s * _SUPER, _SUPER)])
        _drain(buf1, gsem1)
        pltpu.sync_copy(buf1, out_hbm.at[pl.ds(base + (s + 1) * _SUPER, _SUPER)])
        return carry
    lax.fori_loop(0, n_super // 2, _super_pair, 0)


def kernel(input_ids, embedding, offsets):
    B, T, H = input_ids.shape
    R = B * T * H
    rows_per_w = R // _NW
    n_chunks = rows_per_w // _CHUNK

    ids_flat = input_ids.reshape(_NW, n_chunks, _CHUNK).astype(jnp.int32)
    offs16 = jnp.tile(offsets.astype(jnp.int32), 16 // H)

    mesh = plsc.VectorSubcoreMesh(core_axis_name="c", subcore_axis_name="s",
                                  num_cores=_NC, num_subcores=_NS)
    run = pl.kernel(
        _gather_body,
        out_type=jax.ShapeDtypeStruct((R, _D), jnp.float32),
        mesh=mesh,
        scratch_types=[
            pltpu.VMEM((n_chunks, _CHUNK), jnp.int32),
            pltpu.VMEM((16,), jnp.int32),
            pltpu.VMEM((_SUPER, _D), jnp.float32),
            pltpu.VMEM((_SUPER, _D), jnp.float32),
            pltpu.SemaphoreType.DMA,
            pltpu.SemaphoreType.DMA,
        ],
        compiler_params=pltpu.CompilerParams(use_tc_tiling_on_sc=False),
    )
    out = run(ids_flat, offs16, embedding)
    return out.reshape(B, T, H, _D)
